# baseline (speedup 1.0000x reference)
import functools

import jax
import jax.numpy as jnp
from jax.experimental import pallas as pl
from jax.experimental.pallas import tpu as pltpu


def _flash_body(q_ref, k_ref, v_ref, he_ref, ho_ref, o_ref,
                qn_ref, m_ref, d_ref, acc_ref):
    i = pl.program_id(0)
    nsteps = pl.num_programs(0)

    @pl.when(i == 0)
    def _init():
        q = q_ref[...]
        qn = q / jnp.maximum(
            jnp.sqrt(jnp.sum(q * q, axis=1, keepdims=True)), 1e-12)
        qn_ref[...] = qn
        m_ref[...] = jnp.full_like(m_ref, -jnp.inf)
        d_ref[...] = jnp.zeros_like(d_ref)
        acc_ref[...] = jnp.zeros_like(acc_ref)

    d = q_ref.shape[1]
    k2 = k_ref[...]
    qn = qn_ref[...]
    ke = k2[:, :d]
    ko = k2[:, d:]
    ones = jnp.ones((1, d), jnp.float32)

    def scores_half(kh, h_row):
        raw = jax.lax.dot_general(
            qn, kh, (((1,), (1,)), ((), ())),
            preferred_element_type=jnp.float32)
        sumsq = jax.lax.dot_general(
            ones, kh * kh, (((1,), (1,)), ((), ())),
            preferred_element_type=jnp.float32)
        inv_norm = 1.0 / jnp.maximum(jnp.sqrt(sumsq), 1e-12)
        return raw * (h_row * inv_norm * 10.0)

    se = scores_half(ke, he_ref[0])
    so = scores_half(ko, ho_ref[0])

    m_prev = m_ref[...]
    m_cur = jnp.maximum(jnp.max(se, axis=1, keepdims=True),
                        jnp.max(so, axis=1, keepdims=True))
    m_new = jnp.maximum(m_prev, m_cur)
    alpha = jnp.exp(m_prev - m_new)
    pe = jnp.exp(se - m_new)
    po = jnp.exp(so - m_new)
    m_ref[...] = m_new
    d_ref[...] = (d_ref[...] * alpha
                  + jnp.sum(pe, axis=1, keepdims=True)
                  + jnp.sum(po, axis=1, keepdims=True))
    v2 = v_ref[...]
    pv = jax.lax.dot_general(
        pe, v2[:, :d], (((1,), (0,)), ((), ())),
        preferred_element_type=jnp.float32)
    pv += jax.lax.dot_general(
        po, v2[:, d:], (((1,), (0,)), ((), ())),
        preferred_element_type=jnp.float32)
    acc_ref[...] = acc_ref[...] * alpha + pv

    @pl.when(i == nsteps - 1)
    def _done():
        o_ref[...] = acc_ref[...] / d_ref[...]


@functools.partial(jax.jit, static_argnames=("interpret",))
def kernel(latent, keys, vals, hardness, interpret=False):
    b, l, d = latent.shape
    s = keys.shape[0]
    nq = b * l
    q = latent.reshape(nq, d)
    chunk = 10000 if s % 10000 == 0 else s
    c2 = chunk // 2
    grid = (s // chunk,)
    k2 = keys.reshape(s // 2, 2 * d)
    v2 = vals.reshape(s // 2, 2 * d)
    hp = hardness.reshape(s // 2, 2)
    he = hp[:, 0].reshape(s // chunk, 1, c2)
    ho = hp[:, 1].reshape(s // chunk, 1, c2)
    out = pl.pallas_call(
        _flash_body,
        grid=grid,
        in_specs=[
            pl.BlockSpec((nq, d), lambda i: (0, 0)),
            pl.BlockSpec((c2, 2 * d), lambda i: (i, 0)),
            pl.BlockSpec((c2, 2 * d), lambda i: (i, 0)),
            pl.BlockSpec((1, 1, c2), lambda i: (i, 0, 0)),
            pl.BlockSpec((1, 1, c2), lambda i: (i, 0, 0)),
        ],
        out_specs=pl.BlockSpec((nq, d), lambda i: (0, 0)),
        out_shape=jax.ShapeDtypeStruct((nq, d), jnp.float32),
        scratch_shapes=[
            pltpu.VMEM((nq, d), jnp.float32),
            pltpu.VMEM((nq, 1), jnp.float32),
            pltpu.VMEM((nq, 1), jnp.float32),
            pltpu.VMEM((nq, d), jnp.float32),
        ],
        interpret=interpret,
    )(q, k2, v2, he, ho)
    return out.reshape(b, l, d)


# speedup vs baseline: 2.1034x; 2.1034x over previous
import functools

import jax
import jax.numpy as jnp
from jax.experimental import pallas as pl
from jax.experimental.pallas import tpu as pltpu

_NBUF = 4
_LOOKAHEAD = 3


def _flash_body(q_ref, k_hbm, v_hbm, h_ref, o_ref,
                k_buf, v_buf, qn_ref, m_ref, d_ref, acc_ref,
                k_sem, v_sem):
    i = pl.program_id(0)
    nsteps = pl.num_programs(0)

    def start(step):
        b = jax.lax.rem(step, _NBUF)
        pltpu.make_async_copy(k_hbm.at[step], k_buf.at[b], k_sem.at[b]).start()
        pltpu.make_async_copy(v_hbm.at[step], v_buf.at[b], v_sem.at[b]).start()

    @pl.when(i == 0)
    def _init():
        for s in range(_LOOKAHEAD):
            start(s)
        q = q_ref[...]
        qn = q / jnp.maximum(
            jnp.sqrt(jnp.sum(q * q, axis=1, keepdims=True)), 1e-12)
        qn_ref[...] = qn
        m_ref[...] = jnp.full_like(m_ref, -jnp.inf)
        d_ref[...] = jnp.zeros_like(d_ref)
        acc_ref[...] = jnp.zeros_like(acc_ref)

    @pl.when(jnp.logical_and(i > 0, i + _LOOKAHEAD - 1 < nsteps))
    def _prefetch():
        start(i + _LOOKAHEAD - 1)

    b = jax.lax.rem(i, _NBUF)
    pltpu.make_async_copy(k_hbm.at[i], k_buf.at[b], k_sem.at[b]).wait()
    pltpu.make_async_copy(v_hbm.at[i], v_buf.at[b], v_sem.at[b]).wait()

    k = k_buf[b]
    qn = qn_ref[...]
    d = qn.shape[1]
    raw = jax.lax.dot_general(
        qn, k, (((1,), (1,)), ((), ())),
        preferred_element_type=jnp.float32)
    ones = jnp.ones((1, d), jnp.float32)
    sumsq = jax.lax.dot_general(
        ones, k * k, (((1,), (1,)), ((), ())),
        preferred_element_type=jnp.float32)
    inv_norm = 1.0 / jnp.maximum(jnp.sqrt(sumsq), 1e-12)
    scores = raw * (h_ref[0] * inv_norm * 10.0)

    m_prev = m_ref[...]
    m_new = jnp.maximum(m_prev, jnp.max(scores, axis=1, keepdims=True))
    alpha = jnp.exp(m_prev - m_new)
    p = jnp.exp(scores - m_new)
    m_ref[...] = m_new
    d_ref[...] = d_ref[...] * alpha + jnp.sum(p, axis=1, keepdims=True)
    pv = jax.lax.dot_general(
        p, v_buf[b], (((1,), (0,)), ((), ())),
        preferred_element_type=jnp.float32)
    acc_ref[...] = acc_ref[...] * alpha + pv

    @pl.when(i == nsteps - 1)
    def _done():
        o_ref[...] = acc_ref[...] / d_ref[...]


@functools.partial(jax.jit, static_argnames=("interpret",))
def kernel(latent, keys, vals, hardness, interpret=False):
    b, l, d = latent.shape
    s = keys.shape[0]
    nq = b * l
    q = latent.reshape(nq, d)
    chunk = 5000 if s % 5000 == 0 else s
    nsteps = s // chunk
    grid = (nsteps,)
    k3 = keys.reshape(nsteps, chunk, d)
    v3 = vals.reshape(nsteps, chunk, d)
    h3 = hardness.reshape(nsteps, 1, chunk)
    out = pl.pallas_call(
        _flash_body,
        grid=grid,
        in_specs=[
            pl.BlockSpec((nq, d), lambda i: (0, 0)),
            pl.BlockSpec(memory_space=pl.ANY),
            pl.BlockSpec(memory_space=pl.ANY),
            pl.BlockSpec((1, 1, chunk), lambda i: (i, 0, 0)),
        ],
        out_specs=pl.BlockSpec((nq, d), lambda i: (0, 0)),
        out_shape=jax.ShapeDtypeStruct((nq, d), jnp.float32),
        scratch_shapes=[
            pltpu.VMEM((_NBUF, chunk, d), jnp.float32),
            pltpu.VMEM((_NBUF, chunk, d), jnp.float32),
            pltpu.VMEM((nq, d), jnp.float32),
            pltpu.VMEM((nq, 1), jnp.float32),
            pltpu.VMEM((nq, 1), jnp.float32),
            pltpu.VMEM((nq, d), jnp.float32),
            pltpu.SemaphoreType.DMA((_NBUF,)),
            pltpu.SemaphoreType.DMA((_NBUF,)),
        ],
        interpret=interpret,
    )(q, k3, v3, h3)
    return out.reshape(b, l, d)
